# baseline (speedup 1.0000x reference)
import jax
import jax.numpy as jnp
from jax.experimental import pallas as pl
from jax.experimental.pallas import tpu as pltpu

_BH = 4096
_K = 2


def _half_loss(x, r, mu, mu2):
    x2 = jnp.sum(x * x, axis=1, keepdims=True)
    cross = jax.lax.dot_general(
        x, mu,
        dimension_numbers=(((1,), (1,)), ((), ())),
        preferred_element_type=jnp.float32,
    )
    dist2 = jnp.maximum(x2 + mu2 - 2.0 * cross, 0.0)
    return jnp.sum(r * dist2, axis=0, keepdims=True)


def kernel(X, r, mus):
    n, d = X.shape
    m = mus.shape[0]
    g = n // (_K * _BH)
    inv_n = 1.0 / n

    def _loss_body(x0_ref, x1_ref, r0_ref, r1_ref, mu_ref, o_ref, acc_ref):
        j = pl.program_id(0)
        mu = mu_ref[...]
        mu2 = jnp.sum(mu * mu, axis=1, keepdims=True).T
        s = (_half_loss(x0_ref[...], r0_ref[...], mu, mu2)
             + _half_loss(x1_ref[...], r1_ref[...], mu, mu2))

        @pl.when(j == 0)
        def _init():
            acc_ref[...] = s

        @pl.when(j != 0)
        def _accum():
            acc_ref[...] += s

        @pl.when(j == g - 1)
        def _finish():
            o_ref[0, 0] = jnp.sum(acc_ref[...]) * inv_n

    def _spec(k, w):
        return pl.BlockSpec((_BH, w), lambda j, k=k: (_K * j + k, 0))

    in_specs = (
        [_spec(k, d) for k in range(_K)]
        + [_spec(k, m) for k in range(_K)]
        + [pl.BlockSpec((m, d), lambda j: (0, 0))]
    )
    out = pl.pallas_call(
        _loss_body,
        grid=(g,),
        in_specs=in_specs,
        out_specs=pl.BlockSpec(memory_space=pltpu.SMEM),
        out_shape=jax.ShapeDtypeStruct((1, 1), jnp.float32),
        scratch_shapes=[pltpu.VMEM((1, m), jnp.float32)],
        compiler_params=pltpu.CompilerParams(
            dimension_semantics=("arbitrary",),
        ),
    )(X, X, r, r, mus)
    return jnp.reshape(out, ())
